# baseline (speedup 1.0000x reference)
import functools

import jax
import jax.numpy as jnp
from jax import lax
from jax.experimental import pallas as pl
from jax.experimental.pallas import tpu as pltpu
from jax.experimental.pallas import tpu_sc as plsc

NC = 2
NS = 16
NW = NC * NS

FEAT = 64
CH = 128
GROUP = 8


def _sc_gather(idx, table, n_rows):
    nchunk = idx.shape[1]
    ngroup = nchunk // GROUP
    b_per_w = nchunk * CH
    mesh = plsc.VectorSubcoreMesh(core_axis_name="c", subcore_axis_name="s")

    @functools.partial(
        pl.kernel,
        mesh=mesh,
        compiler_params=pltpu.CompilerParams(use_tc_tiling_on_sc=False),
        out_type=jax.ShapeDtypeStruct((n_rows, FEAT), jnp.float32),
        scratch_types=[
            pltpu.VMEM((nchunk, CH), jnp.int32),
            pltpu.VMEM((GROUP, CH, FEAT), jnp.float32),
            pltpu.SemaphoreType.DMA,
            pltpu.SemaphoreType.DMA,
        ],
    )
    def gather_kernel(idx_hbm, table_hbm, out_hbm, idx_v, rows_v, sem_g, sem_s):
        wid = lax.axis_index("s") * NC + lax.axis_index("c")
        base = wid * b_per_w
        pltpu.sync_copy(idx_hbm.at[wid], idx_v)

        def group_body(g, carry):
            c0 = g * GROUP
            for b in range(GROUP):
                pltpu.async_copy(
                    table_hbm.at[idx_v.at[c0 + b]], rows_v.at[b], sem_g)
            for b in range(GROUP):
                pltpu.make_async_copy(
                    table_hbm.at[idx_v.at[c0 + b]], rows_v.at[b], sem_g).wait()
                pltpu.async_copy(
                    rows_v.at[b],
                    out_hbm.at[pl.ds(base + (c0 + b) * CH, CH)], sem_s)
            for b in range(GROUP):
                pltpu.make_async_copy(
                    rows_v.at[b],
                    out_hbm.at[pl.ds(base + (c0 + b) * CH, CH)], sem_s).wait()
            return carry

        lax.fori_loop(0, ngroup, group_body, 0)

    return gather_kernel(idx, table)


def _dense_head(emb, W, b):
    n = emb.shape[0]
    out_dim = W.shape[1]
    bm = 4096

    def mm_kernel(e_ref, w_ref, b_ref, o_ref):
        o_ref[...] = jnp.dot(
            e_ref[...], w_ref[...], preferred_element_type=jnp.float32
        ) + b_ref[...]

    return pl.pallas_call(
        mm_kernel,
        grid=(n // bm,),
        in_specs=[
            pl.BlockSpec((bm, FEAT), lambda i: (i, 0)),
            pl.BlockSpec((FEAT, out_dim), lambda i: (0, 0)),
            pl.BlockSpec((1, out_dim), lambda i: (0, 0)),
        ],
        out_specs=pl.BlockSpec((bm, out_dim), lambda i: (i, 0)),
        out_shape=jax.ShapeDtypeStruct((n, out_dim), jnp.float32),
    )(emb, W, b.reshape(1, out_dim))


def kernel(x, table, W, b):
    batch, hist = x.shape
    n_rows = batch * hist
    idx = x.reshape(NW, n_rows // (NW * CH), CH).astype(jnp.int32)
    emb = _sc_gather(idx, table, n_rows)
    out = _dense_head(emb, W, b)
    return out.reshape(batch, hist, W.shape[1])


# speedup vs baseline: 1.2298x; 1.2298x over previous
import functools

import jax
import jax.numpy as jnp
from jax import lax
from jax.experimental import pallas as pl
from jax.experimental.pallas import tpu as pltpu
from jax.experimental.pallas import tpu_sc as plsc

NC = 2
NS = 16
NW = NC * NS

FEAT = 64
CH = 128
GROUP = 8


def _sc_gather(idx, table, n_rows):
    nchunk = idx.shape[1]
    ngroup = nchunk // GROUP
    b_per_w = nchunk * CH
    mesh = plsc.VectorSubcoreMesh(core_axis_name="c", subcore_axis_name="s")

    @functools.partial(
        pl.kernel,
        mesh=mesh,
        compiler_params=pltpu.CompilerParams(use_tc_tiling_on_sc=False),
        out_type=jax.ShapeDtypeStruct((n_rows, FEAT), jnp.float32),
        scratch_types=[
            pltpu.VMEM((nchunk, CH), jnp.int32),
            pltpu.VMEM((GROUP, CH, FEAT), jnp.float32),
            pltpu.SemaphoreType.DMA,
            pltpu.SemaphoreType.DMA,
        ],
    )
    def gather_kernel(idx_hbm, table_hbm, out_hbm, idx_v, rows_v, sem_g, sem_s):
        wid = lax.axis_index("s") * NC + lax.axis_index("c")
        base = wid * b_per_w
        pltpu.sync_copy(idx_hbm.at[wid], idx_v)

        def group_body(g, carry):
            c0 = g * GROUP
            for b in range(GROUP):
                pltpu.async_copy(
                    table_hbm.at[idx_v.at[c0 + b]], rows_v.at[b], sem_g)
            for b in range(GROUP):
                pltpu.make_async_copy(
                    table_hbm.at[idx_v.at[c0 + b]], rows_v.at[b], sem_g).wait()
                pltpu.async_copy(
                    rows_v.at[b],
                    out_hbm.at[pl.ds(base + (c0 + b) * CH, CH)], sem_s)
            for b in range(GROUP):
                pltpu.make_async_copy(
                    rows_v.at[b],
                    out_hbm.at[pl.ds(base + (c0 + b) * CH, CH)], sem_s).wait()
            return carry

        lax.fori_loop(0, ngroup, group_body, 0)

    return gather_kernel(idx, table)


def _dense_head_t(emb, W, b, hist, batch):
    out_dim = W.shape[1]
    bn = 2048
    nb = batch // bn

    def mm_kernel(e_ref, w_ref, b_ref, o_ref):
        acc = lax.dot_general(
            w_ref[...], e_ref[...],
            dimension_numbers=(((0,), (1,)), ((), ())),
            preferred_element_type=jnp.float32,
        )
        o_ref[0] = acc + b_ref[...]

    return pl.pallas_call(
        mm_kernel,
        grid=(hist, nb),
        in_specs=[
            pl.BlockSpec((bn, FEAT), lambda l, j: (l * nb + j, 0)),
            pl.BlockSpec((FEAT, out_dim), lambda l, j: (0, 0)),
            pl.BlockSpec((out_dim, 1), lambda l, j: (0, 0)),
        ],
        out_specs=pl.BlockSpec((1, out_dim, bn), lambda l, j: (l, 0, j)),
        out_shape=jax.ShapeDtypeStruct((hist, out_dim, batch), jnp.float32),
    )(emb, W, b.reshape(out_dim, 1))


def kernel(x, table, W, b):
    batch, hist = x.shape
    n_rows = batch * hist
    idx = x.T.reshape(NW, n_rows // (NW * CH), CH).astype(jnp.int32)
    emb = _sc_gather(idx, table, n_rows)
    out_t = _dense_head_t(emb, W, b, hist, batch)
    return out_t.transpose(2, 0, 1)


# speedup vs baseline: 1.4907x; 1.2122x over previous
import functools

import jax
import jax.numpy as jnp
from jax import lax
from jax.experimental import pallas as pl
from jax.experimental.pallas import tpu as pltpu
from jax.experimental.pallas import tpu_sc as plsc

NC = 2
NS = 16
NW = NC * NS

FEAT = 64
CH = 128
GROUP = 4


def _sc_gather_pairs(idx, table2, n_rows):
    nchunk = idx.shape[1]
    ngroup = nchunk // GROUP
    b_per_w = nchunk * CH
    mesh = plsc.VectorSubcoreMesh(core_axis_name="c", subcore_axis_name="s")

    @functools.partial(
        pl.kernel,
        mesh=mesh,
        compiler_params=pltpu.CompilerParams(use_tc_tiling_on_sc=True),
        out_type=jax.ShapeDtypeStruct((n_rows, 2 * FEAT), jnp.float32),
        scratch_types=[
            pltpu.VMEM((nchunk, CH), jnp.int32),
            pltpu.VMEM((GROUP, CH, 2 * FEAT), jnp.float32),
            pltpu.SemaphoreType.DMA,
            pltpu.SemaphoreType.DMA,
        ],
    )
    def gather_kernel(idx_hbm, table_hbm, out_hbm, idx_v, rows_v, sem_g, sem_s):
        wid = lax.axis_index("s") * NC + lax.axis_index("c")
        base = wid * b_per_w
        pltpu.sync_copy(idx_hbm.at[wid], idx_v)

        def group_body(g, carry):
            c0 = g * GROUP
            for b in range(GROUP):
                pltpu.async_copy(
                    table_hbm.at[idx_v.at[c0 + b]], rows_v.at[b], sem_g)
            for b in range(GROUP):
                pltpu.make_async_copy(
                    table_hbm.at[idx_v.at[c0 + b]], rows_v.at[b], sem_g).wait()
                pltpu.async_copy(
                    rows_v.at[b],
                    out_hbm.at[pl.ds(base + (c0 + b) * CH, CH)], sem_s)
            for b in range(GROUP):
                pltpu.make_async_copy(
                    rows_v.at[b],
                    out_hbm.at[pl.ds(base + (c0 + b) * CH, CH)], sem_s).wait()
            return carry

        lax.fori_loop(0, ngroup, group_body, 0)

    return gather_kernel(idx, table2)


def _dense_head_t(emb2, par, W, b, hist, batch):
    out_dim = W.shape[1]
    bn = 2048
    nb = batch // bn

    def mm_kernel(e_ref, p_ref, wlo_ref, whi_ref, b_ref, o_ref):
        acc_lo = lax.dot_general(
            wlo_ref[...], e_ref[...],
            dimension_numbers=(((0,), (1,)), ((), ())),
            preferred_element_type=jnp.float32)
        acc_hi = lax.dot_general(
            whi_ref[...], e_ref[...],
            dimension_numbers=(((0,), (1,)), ((), ())),
            preferred_element_type=jnp.float32)
        sel = jnp.where(p_ref[0] != 0.0, acc_hi, acc_lo)
        o_ref[0] = sel + b_ref[...]

    return pl.pallas_call(
        mm_kernel,
        grid=(hist, nb),
        in_specs=[
            pl.BlockSpec((bn, 2 * FEAT), lambda l, j: (l * nb + j, 0)),
            pl.BlockSpec((1, 1, bn), lambda l, j: (l, 0, j)),
            pl.BlockSpec((2 * FEAT, out_dim), lambda l, j: (0, 0)),
            pl.BlockSpec((2 * FEAT, out_dim), lambda l, j: (0, 0)),
            pl.BlockSpec((out_dim, 1), lambda l, j: (0, 0)),
        ],
        out_specs=pl.BlockSpec((1, out_dim, bn), lambda l, j: (l, 0, j)),
        out_shape=jax.ShapeDtypeStruct((hist, out_dim, batch), jnp.float32),
    )(emb2, par.reshape(hist, 1, batch), *_interleaved_ws(W), b.reshape(out_dim, 1))


def _interleaved_ws(W):
    wz = jnp.zeros_like(W)
    w_lo = jnp.stack([W, wz], axis=1).reshape(2 * FEAT, W.shape[1])
    w_hi = jnp.stack([wz, W], axis=1).reshape(2 * FEAT, W.shape[1])
    return w_lo, w_hi


def kernel(x, table, W, b):
    batch, hist = x.shape
    n_rows = batch * hist
    xt = x.T.astype(jnp.int32)
    half = table.shape[0] // 2
    idx = jnp.where(xt >= half, xt - half, xt).reshape(
        NW, n_rows // (NW * CH), CH)
    par = (xt >= half).astype(jnp.float32)
    table2 = table.T.reshape(2 * FEAT, half).T
    emb2 = _sc_gather_pairs(idx, table2, n_rows)
    out_t = _dense_head_t(emb2, par, W, b, hist, batch)
    return out_t.transpose(2, 0, 1)


# speedup vs baseline: 1.5634x; 1.0487x over previous
import functools

import jax
import jax.numpy as jnp
from jax import lax
from jax.experimental import pallas as pl
from jax.experimental.pallas import tpu as pltpu
from jax.experimental.pallas import tpu_sc as plsc

NC = 2
NS = 16
NW = NC * NS

FEAT = 64
CH = 128
GROUP = 4


def _sc_gather_pairs(idx, table2, n_rows):
    nchunk = idx.shape[1]
    ngroup = nchunk // GROUP
    b_per_w = nchunk * CH
    mesh = plsc.VectorSubcoreMesh(core_axis_name="c", subcore_axis_name="s")

    @functools.partial(
        pl.kernel,
        mesh=mesh,
        compiler_params=pltpu.CompilerParams(use_tc_tiling_on_sc=True),
        out_type=jax.ShapeDtypeStruct((n_rows, 2 * FEAT), jnp.float32),
        scratch_types=[
            pltpu.VMEM((nchunk, CH), jnp.int32),
            pltpu.VMEM((GROUP, CH, 2 * FEAT), jnp.float32),
            pltpu.SemaphoreType.DMA,
            pltpu.SemaphoreType.DMA,
        ],
    )
    def gather_kernel(idx_hbm, table_hbm, out_hbm, idx_v, rows_v, sem_g, sem_s):
        wid = lax.axis_index("s") * NC + lax.axis_index("c")
        base = wid * b_per_w
        pltpu.sync_copy(idx_hbm.at[wid], idx_v)

        def group_body(g, carry):
            c0 = g * GROUP
            for b in range(GROUP):
                pltpu.async_copy(
                    table_hbm.at[idx_v.at[c0 + b]], rows_v.at[b], sem_g)
            for b in range(GROUP):
                pltpu.make_async_copy(
                    table_hbm.at[idx_v.at[c0 + b]], rows_v.at[b], sem_g).wait()
                pltpu.async_copy(
                    rows_v.at[b],
                    out_hbm.at[pl.ds(base + (c0 + b) * CH, CH)], sem_s)
            for b in range(GROUP):
                pltpu.make_async_copy(
                    rows_v.at[b],
                    out_hbm.at[pl.ds(base + (c0 + b) * CH, CH)], sem_s).wait()
            return carry

        lax.fori_loop(0, ngroup, group_body, 0)

    return gather_kernel(idx, table2)


def _fold_table(table_t, vocab):
    bn = 2048
    grid = (vocab + bn - 1) // bn

    def fold_kernel(i_ref, o_ref):
        tt = i_ref[...].T
        t3 = tt.reshape(bn // 16, 16, FEAT)
        cat = jnp.concatenate([t3[:, 0:8, :], t3[:, 8:16, :]], axis=2)
        o_ref[...] = cat.reshape(bn // 2, 2 * FEAT)

    return pl.pallas_call(
        fold_kernel,
        grid=(grid,),
        in_specs=[pl.BlockSpec((FEAT, bn), lambda j: (0, j))],
        out_specs=pl.BlockSpec((bn // 2, 2 * FEAT), lambda j: (j, 0)),
        out_shape=jax.ShapeDtypeStruct((vocab // 2, 2 * FEAT), jnp.float32),
    )(table_t)


def _dense_head_t(emb2, par, W, b, hist, batch):
    out_dim = W.shape[1]
    bn = 2048
    nb = batch // bn

    def mm_kernel(e_ref, p_ref, wlo_ref, whi_ref, b_ref, o_ref):
        acc_lo = lax.dot_general(
            wlo_ref[...], e_ref[...],
            dimension_numbers=(((0,), (1,)), ((), ())),
            preferred_element_type=jnp.float32)
        acc_hi = lax.dot_general(
            whi_ref[...], e_ref[...],
            dimension_numbers=(((0,), (1,)), ((), ())),
            preferred_element_type=jnp.float32)
        sel = jnp.where(p_ref[0] != 0.0, acc_hi, acc_lo)
        o_ref[0] = sel + b_ref[...]

    return pl.pallas_call(
        mm_kernel,
        grid=(hist, nb),
        in_specs=[
            pl.BlockSpec((bn, 2 * FEAT), lambda l, j: (l * nb + j, 0)),
            pl.BlockSpec((1, 1, bn), lambda l, j: (l, 0, j)),
            pl.BlockSpec((2 * FEAT, out_dim), lambda l, j: (0, 0)),
            pl.BlockSpec((2 * FEAT, out_dim), lambda l, j: (0, 0)),
            pl.BlockSpec((out_dim, 1), lambda l, j: (0, 0)),
        ],
        out_specs=pl.BlockSpec((1, out_dim, bn), lambda l, j: (l, 0, j)),
        out_shape=jax.ShapeDtypeStruct((hist, out_dim, batch), jnp.float32),
    )(emb2, par.reshape(hist, 1, batch), *_interleaved_ws(W), b.reshape(out_dim, 1))


def _interleaved_ws(W):
    wz = jnp.zeros_like(W)
    w_lo = jnp.concatenate([W, wz], axis=0)
    w_hi = jnp.concatenate([wz, W], axis=0)
    return w_lo, w_hi


def kernel(x, table, W, b):
    batch, hist = x.shape
    n_rows = batch * hist
    xt = x.T.astype(jnp.int32)
    idx = (((xt >> 4) << 3) | (xt & 7)).reshape(NW, n_rows // (NW * CH), CH)
    par = ((xt >> 3) & 1).astype(jnp.float32)
    table2 = _fold_table(table.T, table.shape[0])
    emb2 = _sc_gather_pairs(idx, table2, n_rows)
    out_t = _dense_head_t(emb2, par, W, b, hist, batch)
    return out_t.transpose(2, 0, 1)
